# baseline (speedup 1.0000x reference)
import functools
import jax
import jax.numpy as jnp
from jax import lax
from jax.experimental import pallas as pl
from jax.experimental.pallas import tpu as pltpu
from jax.experimental.pallas import tpu_sc as plsc

D = 64
SCALE = 8.0
L = 16
NBUF = 4
LEAD = 2
BLK = 128


def _make_emb_kernel(n_rows: int, row_len: int, num_cores: int):
    n_tc = n_rows // BLK
    mesh = plsc.VectorSubcoreMesh(core_axis_name="c", subcore_axis_name="s")

    @functools.partial(
        pl.kernel,
        out_type=jax.ShapeDtypeStruct((row_len, D // 8, n_tc, 8, BLK),
                                      jnp.float32),
        mesh=mesh,
        scratch_types=[
            pltpu.VMEM((BLK, row_len), jnp.int32),
            pltpu.VMEM((NBUF, BLK), jnp.int32),
            pltpu.VMEM((NBUF, BLK, D), jnp.float32),
            pltpu.VMEM((NBUF, D // 8, 8, BLK), jnp.float32),
            pltpu.SemaphoreType.DMA((NBUF,)),
            pltpu.SemaphoreType.DMA((NBUF,)),
        ],
        compiler_params=pltpu.CompilerParams(use_tc_tiling_on_sc=False,
                                             needs_layout_passes=False),
    )
    def _emb(x_hbm, t_hbm, out_hbm, xv, qv, rows, obuf, gsem, osem):
        wid = lax.axis_index("s") * num_cores + lax.axis_index("c")
        pltpu.sync_copy(x_hbm.at[pl.ds(wid * BLK, BLK)], xv)
        iota = lax.iota(jnp.int32, L)
        rowvs = tuple(iota + (g * L) for g in range(BLK // L))

        def prep(j, b):
            jv = jnp.full((L,), j, jnp.int32)
            for g in range(BLK // L):
                qv[b, pl.ds(g * L, L)] = plsc.load_gather(xv, [rowvs[g], jv])

        def g_desc(b):
            return (t_hbm.at[qv.at[b]], rows.at[b], gsem.at[b])

        def o_desc(j, b):
            return (obuf.at[b], out_hbm.at[j, :, wid], osem.at[b])

        def transpose(b):
            src = rows.at[b]
            for k in range(D):
                kv = jnp.full((L,), k, jnp.int32)
                for g in range(BLK // L):
                    v = plsc.load_gather(src, [rowvs[g], kv])
                    obuf[b, k // 8, k % 8, pl.ds(g * L, L)] = v * SCALE

        for j in range(LEAD):
            prep(j, j)
            pltpu.async_copy(*g_desc(j))

        def chunk_body(j, carry):
            b = j & (NBUF - 1)
            pltpu.make_async_copy(*g_desc(b)).wait()

            @pl.when(j >= NBUF)
            def _():
                pltpu.make_async_copy(*o_desc(j - NBUF, b)).wait()

            transpose(b)
            pltpu.async_copy(*o_desc(j, b))

            @pl.when(j + LEAD < row_len)
            def _():
                nb = (j + LEAD) & (NBUF - 1)
                prep(j + LEAD, nb)
                pltpu.async_copy(*g_desc(nb))

            return carry

        lax.fori_loop(0, row_len, chunk_body, 0)

        for b in range(NBUF):
            pltpu.make_async_copy(*o_desc(row_len - NBUF + b, b)).wait()

    return _emb


@jax.jit
def _kernel_impl(x, table):
    info = plsc.get_sparse_core_info()
    n_rows, row_len = x.shape
    emb = _make_emb_kernel(n_rows, row_len, info.num_cores)
    p5 = emb(x.astype(jnp.int32), table)
    t = jnp.transpose(p5, (2, 4, 0, 1, 3))
    return jnp.reshape(t, (n_rows, row_len, D))


_DEBUG_ONCE = []


def _debug_report(x, table):
    if _DEBUG_ONCE:
        return
    _DEBUG_ONCE.append(1)
    import sys
    import re
    try:
        hlo = jax.jit(_kernel_impl.__wrapped__).lower(x, table).compile().as_text()
        for line in hlo.splitlines():
            if re.search(r"bitcast\(|copy\(|reshape\(|ROOT", line):
                print("DBG:", line.strip()[:170], file=sys.stderr)
    except Exception as e:
        print("DBG fail:", repr(e), file=sys.stderr)


def kernel(x, table):
    _debug_report(x, table)
    return _kernel_impl(x, table)


# speedup vs baseline: 1.7439x; 1.7439x over previous
import functools
import jax
import jax.numpy as jnp
from jax import lax
from jax.experimental import pallas as pl
from jax.experimental.pallas import tpu as pltpu
from jax.experimental.pallas import tpu_sc as plsc

D = 64
SCALE = 8.0
L = 16
NBUF = 4
LEAD = 2
BLK = 128


def _make_emb_kernel(n_rows: int, row_len: int, num_cores: int):
    n_tc = n_rows // BLK
    mesh = plsc.VectorSubcoreMesh(core_axis_name="c", subcore_axis_name="s")

    @functools.partial(
        pl.kernel,
        out_type=jax.ShapeDtypeStruct((row_len, D // 8, n_tc, 8, BLK),
                                      jnp.float32),
        mesh=mesh,
        scratch_types=[
            pltpu.VMEM((BLK, row_len), jnp.int32),
            pltpu.VMEM((NBUF, BLK), jnp.int32),
            pltpu.VMEM((NBUF, BLK, D), jnp.float32),
            pltpu.VMEM((NBUF, D // 8, 8, BLK), jnp.float32),
            pltpu.SemaphoreType.DMA((NBUF,)),
            pltpu.SemaphoreType.DMA((NBUF,)),
        ],
        compiler_params=pltpu.CompilerParams(use_tc_tiling_on_sc=False,
                                             needs_layout_passes=False),
    )
    def _emb(x_hbm, t_hbm, out_hbm, xv, qv, rows, obuf, gsem, osem):
        wid = lax.axis_index("s") * num_cores + lax.axis_index("c")
        pltpu.sync_copy(x_hbm.at[pl.ds(wid * BLK, BLK)], xv)
        iota = lax.iota(jnp.int32, L)
        rowvs = tuple(iota + (g * L) for g in range(BLK // L))

        def prep(j, b):
            jv = jnp.full((L,), j, jnp.int32)
            for g in range(BLK // L):
                qv[b, pl.ds(g * L, L)] = plsc.load_gather(xv, [rowvs[g], jv])

        def g_desc(b):
            return (t_hbm.at[qv.at[b]], rows.at[b], gsem.at[b])

        def o_desc(j, b):
            return (obuf.at[b], out_hbm.at[j, :, wid], osem.at[b])

        def transpose(b):
            src = rows.at[b]
            dst = obuf.at[b]

            def d_body(d, carry):
                perm = (iota + d) & (L - 1)
                for q in range(D // L):
                    kv = perm + (L * q)
                    trv = lax.shift_right_logical(kv, 3)
                    k8v = kv & 7
                    for g in range(BLK // L):
                        v = plsc.load_gather(src, [rowvs[g], kv])
                        plsc.store_scatter(dst, [trv, k8v, rowvs[g]], v * SCALE)
                return carry

            lax.fori_loop(0, L, d_body, 0)

        for j in range(LEAD):
            prep(j, j)
            pltpu.async_copy(*g_desc(j))

        def chunk_body(j, carry):
            b = j & (NBUF - 1)
            pltpu.make_async_copy(*g_desc(b)).wait()

            @pl.when(j >= NBUF)
            def _():
                pltpu.make_async_copy(*o_desc(j - NBUF, b)).wait()

            transpose(b)
            pltpu.async_copy(*o_desc(j, b))

            @pl.when(j + LEAD < row_len)
            def _():
                nb = (j + LEAD) & (NBUF - 1)
                prep(j + LEAD, nb)
                pltpu.async_copy(*g_desc(nb))

            return carry

        lax.fori_loop(0, row_len, chunk_body, 0)

        for b in range(NBUF):
            pltpu.make_async_copy(*o_desc(row_len - NBUF + b, b)).wait()

    return _emb


@jax.jit
def _kernel_impl(x, table):
    info = plsc.get_sparse_core_info()
    n_rows, row_len = x.shape
    emb = _make_emb_kernel(n_rows, row_len, info.num_cores)
    p5 = emb(x.astype(jnp.int32), table)
    t = jnp.transpose(p5, (2, 4, 0, 1, 3))
    return jnp.reshape(t, (n_rows, row_len, D))


_DEBUG_ONCE = []


def _debug_report(x, table):
    if _DEBUG_ONCE:
        return
    _DEBUG_ONCE.append(1)
    import sys
    import re
    try:
        hlo = jax.jit(_kernel_impl.__wrapped__).lower(x, table).compile().as_text()
        for line in hlo.splitlines():
            if re.search(r"bitcast\(|copy\(|reshape\(|ROOT", line):
                print("DBG:", line.strip()[:170], file=sys.stderr)
    except Exception as e:
        print("DBG fail:", repr(e), file=sys.stderr)


def kernel(x, table):
    _debug_report(x, table)
    return _kernel_impl(x, table)


# speedup vs baseline: 1.8213x; 1.0444x over previous
import functools
import jax
import jax.numpy as jnp
from jax import lax
from jax.experimental import pallas as pl
from jax.experimental.pallas import tpu as pltpu
from jax.experimental.pallas import tpu_sc as plsc

D = 64
SCALE = 8.0
L = 16
NBUF = 4
LEAD = 2
BLK = 128


def _make_emb_kernel(n_rows: int, row_len: int, num_cores: int):
    n_tc = n_rows // BLK
    mesh = plsc.VectorSubcoreMesh(core_axis_name="c", subcore_axis_name="s")

    @functools.partial(
        pl.kernel,
        out_type=jax.ShapeDtypeStruct((row_len, D // 8, n_tc, 8, BLK),
                                      jnp.float32),
        mesh=mesh,
        scratch_types=[
            pltpu.VMEM((BLK, row_len), jnp.int32),
            pltpu.VMEM((NBUF, BLK), jnp.int32),
            pltpu.VMEM((NBUF, BLK, D), jnp.float32),
            pltpu.VMEM((NBUF, D // 8, 8, BLK), jnp.float32),
            pltpu.SemaphoreType.DMA((NBUF,)),
            pltpu.SemaphoreType.DMA((NBUF,)),
        ],
        compiler_params=pltpu.CompilerParams(use_tc_tiling_on_sc=False,
                                             needs_layout_passes=False),
    )
    def _emb(x_hbm, t_hbm, out_hbm, xv, qv, rows, obuf, gsem, osem):
        wid = lax.axis_index("s") * num_cores + lax.axis_index("c")
        pltpu.sync_copy(x_hbm.at[pl.ds(wid * BLK, BLK)], xv)
        iota = lax.iota(jnp.int32, L)
        rowvs = tuple(iota + (g * L) for g in range(BLK // L))

        def prep(j, b):
            jv = jnp.full((L,), j, jnp.int32)
            for g in range(BLK // L):
                qv[b, pl.ds(g * L, L)] = plsc.load_gather(xv, [rowvs[g], jv])

        def g_desc(b):
            return (t_hbm.at[qv.at[b]], rows.at[b], gsem.at[b])

        def o_desc(j, b):
            return (obuf.at[b], out_hbm.at[j, :, wid], osem.at[b])

        def transpose(b):
            src = rows.at[b]
            dst = obuf.at[b]

            def d_body(d, carry):
                perm = (iota + d) & (L - 1)
                for q in range(D // L):
                    kv = perm + (L * q)
                    trv = lax.shift_right_logical(kv, 3)
                    k8v = kv & 7
                    for g in range(BLK // L):
                        v = plsc.load_gather(src, [rowvs[g], kv])
                        plsc.store_scatter(dst, [trv, k8v, rowvs[g]], v * SCALE)
                return carry

            lax.fori_loop(0, L, d_body, 0, unroll=4)

        for j in range(LEAD):
            prep(j, j)
            pltpu.async_copy(*g_desc(j))

        def chunk_body(j, carry):
            b = j & (NBUF - 1)
            pltpu.make_async_copy(*g_desc(b)).wait()

            @pl.when(j >= NBUF)
            def _():
                pltpu.make_async_copy(*o_desc(j - NBUF, b)).wait()

            transpose(b)
            pltpu.async_copy(*o_desc(j, b))

            @pl.when(j + LEAD < row_len)
            def _():
                nb = (j + LEAD) & (NBUF - 1)
                prep(j + LEAD, nb)
                pltpu.async_copy(*g_desc(nb))

            return carry

        lax.fori_loop(0, row_len, chunk_body, 0)

        for b in range(NBUF):
            pltpu.make_async_copy(*o_desc(row_len - NBUF + b, b)).wait()

    return _emb


@jax.jit
def _kernel_impl(x, table):
    info = plsc.get_sparse_core_info()
    n_rows, row_len = x.shape
    emb = _make_emb_kernel(n_rows, row_len, info.num_cores)
    p5 = emb(x.astype(jnp.int32), table)
    t = jnp.transpose(p5, (2, 4, 0, 1, 3))
    return jnp.reshape(t, (n_rows, row_len, D))


_DEBUG_ONCE = []


def _debug_report(x, table):
    if _DEBUG_ONCE:
        return
    _DEBUG_ONCE.append(1)
    import sys
    import re
    try:
        hlo = jax.jit(_kernel_impl.__wrapped__).lower(x, table).compile().as_text()
        for line in hlo.splitlines():
            if re.search(r"bitcast\(|copy\(|reshape\(|ROOT", line):
                print("DBG:", line.strip()[:170], file=sys.stderr)
    except Exception as e:
        print("DBG fail:", repr(e), file=sys.stderr)


def kernel(x, table):
    _debug_report(x, table)
    return _kernel_impl(x, table)
